# baseline (speedup 1.0000x reference)
import jax
import jax.numpy as jnp
from jax import lax
from jax.experimental import pallas as pl
from jax.experimental.pallas import tpu as pltpu
from jax.experimental.pallas import tpu_sc as plsc

INPUT_DIM = 1024
OUTPUT_DIM = 1024
K_DIM = 512
HALF = K_DIM // 2
HEADS = 4
KNN = 16
N_KEYS = 256
SIZE = N_KEYS * N_KEYS
V_DIM = OUTPUT_DIM
TOKENS = 2048

TOK_BLK = 256
GRID = TOKENS // TOK_BLK
NWORK = 32
TPW = TOKENS // NWORK
NSEL = HEADS * KNN
QROWS = 32
QPT = NSEL // QROWS
NQ = TPW * QPT
OUTB = 16
INT_MIN = -2147483648

_GDN = lax.GatherDimensionNumbers(
    offset_dims=(), collapsed_slice_dims=(0,), start_index_map=(0,))


def _splat(vec, lane):
    idx = jnp.full((16, 1), lane, jnp.int32)
    return lax.gather(vec, idx, dimension_numbers=_GDN, slice_sizes=(1,),
                      mode=lax.GatherScatterMode.PROMISE_IN_BOUNDS)


_SCALE = float(1 << 20)


def _pack(s, lane):
    ki = jnp.round(s * _SCALE).astype(jnp.int32)
    return lax.shift_left(ki, 8) | lane


def _unpack(p):
    return lax.shift_right_arithmetic(p, 8).astype(jnp.float32) * (1.0 / _SCALE)


def _topk16v(pk):
    ms = []
    for _ in range(KNN):
        m = jnp.max(pk, axis=0)
        ms.append(m)
        pk = jnp.where(pk == m[None, :], jnp.int32(INT_MIN), pk)
    return jnp.stack(ms, axis=0)


def _rowpick(sel, tab):
    kio = lax.broadcasted_iota(jnp.int32, (KNN, KNN, sel.shape[1]), 1)
    hits = jnp.where(sel[:, None, :] == kio, tab[None, :, :], 0)
    return jnp.sum(hits, axis=1)


def _select_body(x_ref, wq_ref, bq_ref, keys_ref, idxT_ref, wT_ref):
    qT = lax.dot_general(wq_ref[...], x_ref[...],
                         dimension_numbers=(((1,), (1,)), ((), ())),
                         preferred_element_type=jnp.float32)
    qT = qT + bq_ref[...]
    iota = lax.broadcasted_iota(jnp.int32, (N_KEYS, TOK_BLK), 0)
    for h in range(HEADS):
        q1 = qT[h * K_DIM:h * K_DIM + HALF, :]
        q2 = qT[h * K_DIM + HALF:(h + 1) * K_DIM, :]
        s1 = jnp.dot(keys_ref[2 * h], q1, preferred_element_type=jnp.float32)
        s2 = jnp.dot(keys_ref[2 * h + 1], q2,
                     preferred_element_type=jnp.float32)
        p1 = _topk16v(_pack(s1, iota))
        p2 = _topk16v(_pack(s2, iota))
        i1 = p1 & 255
        i2 = p2 & 255
        v1 = _unpack(p1)
        v2 = _unpack(p2)
        comb = (v1[:, None, :] + v2[None, :, :]).reshape(KNN * KNN, TOK_BLK)
        pc = _topk16v(_pack(comb, iota))
        c = pc & 255
        v = _unpack(pc)
        m = jnp.max(v, axis=0, keepdims=True)
        e = jnp.exp(v - m)
        w = e / jnp.sum(e, axis=0, keepdims=True)
        fin = _rowpick(c >> 4, i1) * N_KEYS + _rowpick(c & 15, i2)
        idxT_ref[:, h * KNN:(h + 1) * KNN] = fin.T
        wT_ref[:, h * KNN:(h + 1) * KNN] = w.T


def _out_body(x_ref, bag_ref, wswT_ref, bsw_ref, wvpT_ref, bvp_ref, o_ref):
    x = x_ref[...]
    g = jnp.dot(x, wswT_ref[...], preferred_element_type=jnp.float32)
    g = g + bsw_ref[...]
    g = g / (1.0 + jnp.exp(-g)) * bag_ref[...]
    o_ref[...] = jnp.dot(g, wvpT_ref[...],
                         preferred_element_type=jnp.float32) + bvp_ref[...]


JU = 4


def _accum_chunk(q, buf, w_v, out_v):
    tok = q >> 1
    row = tok & (OUTB - 1)
    r = q & 1

    def cg_body(cg, carry):
        accs0 = tuple(out_v[row, pl.ds(cg * 256 + ci * 16, 16)]
                      for ci in range(16))

        def j_body(jj, accs):
            accs = list(accs)
            for u in range(JU):
                j = jj * JU + u
                wchunk = w_v[tok, pl.ds(r * QROWS + (j >> 4) * 16, 16)]
                sp = _splat(wchunk, j & 15)
                for ci in range(16):
                    accs[ci] = accs[ci] + sp * buf[
                        j, pl.ds(cg * 256 + ci * 16, 16)]
            return tuple(accs)

        accs = lax.fori_loop(0, QROWS // JU, j_body, accs0)
        for ci in range(16):
            out_v[row, pl.ds(cg * 256 + ci * 16, 16)] = accs[ci]
        return carry

    lax.fori_loop(0, V_DIM // 256, cg_body, 0)


def _zero_out(out_v):
    def zero_body(t, carry):
        for ci in range(V_DIM // 16):
            out_v[t, pl.ds(ci * 16, 16)] = jnp.zeros((16,), jnp.float32)
        return carry

    lax.fori_loop(0, OUTB, zero_body, 0)


def _make_bag_body(tpw, nq):
    def _bag_body(values_hbm, idx_hbm, w_hbm, out_hbm, idx_v, w_v, bufa,
                  bufb, out_v, sema, semb):
        cid = lax.axis_index("c")
        sid = lax.axis_index("s")
        wid = sid * 2 + cid
        base = wid * tpw
        pltpu.sync_copy(idx_hbm.at[pl.ds(base * QPT, nq)], idx_v)
        pltpu.sync_copy(w_hbm.at[pl.ds(base, tpw)], w_v)
        _zero_out(out_v)

        def _start(q, buf, sem):
            for hh in range(2):
                pltpu.make_async_copy(
                    values_hbm.at[idx_v.at[q, hh]],
                    buf.at[pl.ds(hh * (QROWS // 2), QROWS // 2)],
                    sem).start()

        def _wait(q, buf, sem):
            for hh in range(2):
                pltpu.make_async_copy(
                    values_hbm.at[idx_v.at[q, hh]],
                    buf.at[pl.ds(hh * (QROWS // 2), QROWS // 2)],
                    sem).wait()

        _start(0, bufa, sema)
        _start(1, bufb, semb)

        def pair_body(qq, carry):
            q0 = qq * 2
            q1 = q0 + 1
            n0 = lax.rem(q0 + 2, nq)
            n1 = lax.rem(q1 + 2, nq)
            _wait(q0, bufa, sema)
            _accum_chunk(q0, bufa, w_v, out_v)
            _start(n0, bufa, sema)
            _wait(q1, bufb, semb)
            _accum_chunk(q1, bufb, w_v, out_v)
            _start(n1, bufb, semb)

            @pl.when(lax.rem(qq, OUTB) == OUTB - 1)
            def _flush():
                off = pl.multiple_of(base + qq - (OUTB - 1), OUTB)
                pltpu.sync_copy(out_v, out_hbm.at[pl.ds(off, OUTB)])
                _zero_out(out_v)

            return carry

        lax.fori_loop(0, nq // 2, pair_body, 0)
        _wait(0, bufa, sema)
        _wait(1, bufb, semb)

    return _bag_body


def _bag_call(values, idxq, w, ntok):
    tpw = ntok // NWORK
    nq = tpw * QPT
    mesh = plsc.VectorSubcoreMesh(core_axis_name="c", subcore_axis_name="s")
    return pl.kernel(
        _make_bag_body(tpw, nq),
        mesh=mesh,
        out_type=jax.ShapeDtypeStruct((ntok, V_DIM), jnp.float32),
        scratch_types=[
            pltpu.VMEM((nq, 2, QROWS // 2), jnp.int32),
            pltpu.VMEM((tpw, NSEL), jnp.float32),
            pltpu.VMEM((QROWS, V_DIM), jnp.float32),
            pltpu.VMEM((QROWS, V_DIM), jnp.float32),
            pltpu.VMEM((OUTB, V_DIM), jnp.float32),
            pltpu.SemaphoreType.DMA,
            pltpu.SemaphoreType.DMA,
        ],
    )(values, idxq, w)


def _select_call(x, Wq, bq2, keysN, ntok):
    grid = ntok // TOK_BLK
    return pl.pallas_call(
        _select_body,
        grid=(grid,),
        in_specs=[
            pl.BlockSpec((TOK_BLK, INPUT_DIM), lambda i: (i, 0)),
            pl.BlockSpec((HEADS * K_DIM, INPUT_DIM), lambda i: (0, 0)),
            pl.BlockSpec((HEADS * K_DIM, 1), lambda i: (0, 0)),
            pl.BlockSpec((2 * HEADS, N_KEYS, HALF), lambda i: (0, 0, 0)),
        ],
        out_specs=[
            pl.BlockSpec((TOK_BLK, NSEL), lambda i: (i, 0)),
            pl.BlockSpec((TOK_BLK, NSEL), lambda i: (i, 0)),
        ],
        out_shape=[
            jax.ShapeDtypeStruct((ntok, NSEL), jnp.int32),
            jax.ShapeDtypeStruct((ntok, NSEL), jnp.float32),
        ],
    )(x, Wq, bq2, keysN)


def _out_call(x, bag, WswT, bsw2, WvpT, bvp2, ntok):
    grid = ntok // TOK_BLK
    return pl.pallas_call(
        _out_body,
        grid=(grid,),
        in_specs=[
            pl.BlockSpec((TOK_BLK, INPUT_DIM), lambda i: (i, 0)),
            pl.BlockSpec((TOK_BLK, V_DIM), lambda i: (i, 0)),
            pl.BlockSpec((INPUT_DIM, V_DIM), lambda i: (0, 0)),
            pl.BlockSpec((1, V_DIM), lambda i: (0, 0)),
            pl.BlockSpec((V_DIM, OUTPUT_DIM), lambda i: (0, 0)),
            pl.BlockSpec((1, OUTPUT_DIM), lambda i: (0, 0)),
        ],
        out_specs=pl.BlockSpec((TOK_BLK, OUTPUT_DIM), lambda i: (i, 0)),
        out_shape=jax.ShapeDtypeStruct((ntok, OUTPUT_DIM), jnp.float32),
    )(x, bag, WswT, bsw2, WvpT, bvp2)


@jax.jit
def kernel(x, keys, values, Wq, bq, Wvp, bvp, Wsw, bsw):
    keysN = keys.reshape(2 * HEADS, N_KEYS, HALF)
    bq2 = bq.reshape(HEADS * K_DIM, 1)
    WswT = Wsw.T
    WvpT = Wvp.T
    bsw2 = bsw.reshape(1, V_DIM)
    bvp2 = bvp.reshape(1, OUTPUT_DIM)
    htok = TOKENS // 2

    outs = []
    bags = [None, None]
    idxqs = [None, None]
    ws = [None, None]
    for hf in range(2):
        idx, wsel = _select_call(
            x[hf * htok:(hf + 1) * htok], Wq, bq2, keysN, htok)
        idxqs[hf] = idx.reshape(htok * NSEL // QROWS, 2, QROWS // 2)
        ws[hf] = wsel
    for hf in range(2):
        bags[hf] = _bag_call(values, idxqs[hf], ws[hf], htok)
    for hf in range(2):
        outs.append(_out_call(
            x[hf * htok:(hf + 1) * htok], bags[hf], WswT, bsw2, WvpT, bvp2,
            htok))
    return jnp.concatenate(outs, axis=0)


# speedup vs baseline: 1.0172x; 1.0172x over previous
import jax
import jax.numpy as jnp
from jax import lax
from jax.experimental import pallas as pl
from jax.experimental.pallas import tpu as pltpu
from jax.experimental.pallas import tpu_sc as plsc

INPUT_DIM = 1024
OUTPUT_DIM = 1024
K_DIM = 512
HALF = K_DIM // 2
HEADS = 4
KNN = 16
N_KEYS = 256
SIZE = N_KEYS * N_KEYS
V_DIM = OUTPUT_DIM
TOKENS = 2048

TOK_BLK = 256
GRID = TOKENS // TOK_BLK
NWORK = 32
TPW = TOKENS // NWORK
NSEL = HEADS * KNN
QROWS = 32
QPT = NSEL // QROWS
NQ = TPW * QPT
OUTB = 16
INT_MIN = -2147483648

_GDN = lax.GatherDimensionNumbers(
    offset_dims=(), collapsed_slice_dims=(0,), start_index_map=(0,))


def _splat(vec, lane):
    idx = jnp.full((16, 1), lane, jnp.int32)
    return lax.gather(vec, idx, dimension_numbers=_GDN, slice_sizes=(1,),
                      mode=lax.GatherScatterMode.PROMISE_IN_BOUNDS)


_SCALE = float(1 << 20)


def _pack(s, lane):
    ki = jnp.round(s * _SCALE).astype(jnp.int32)
    return lax.shift_left(ki, 8) | lane


def _unpack(p):
    return lax.shift_right_arithmetic(p, 8).astype(jnp.float32) * (1.0 / _SCALE)


def _topk16v(pk):
    ms = []
    for _ in range(KNN):
        m = jnp.max(pk, axis=0)
        ms.append(m)
        pk = jnp.where(pk == m[None, :], jnp.int32(INT_MIN), pk)
    return jnp.stack(ms, axis=0)


def _rowpick(sel, tab):
    kio = lax.broadcasted_iota(jnp.int32, (KNN, KNN, sel.shape[1]), 1)
    hits = jnp.where(sel[:, None, :] == kio, tab[None, :, :], 0)
    return jnp.sum(hits, axis=1)


def _select_body(x_ref, wq_ref, bq_ref, keys_ref, idxT_ref, wT_ref):
    qT = lax.dot_general(wq_ref[...], x_ref[...],
                         dimension_numbers=(((1,), (1,)), ((), ())),
                         preferred_element_type=jnp.float32)
    qT = qT + bq_ref[...]
    iota = lax.broadcasted_iota(jnp.int32, (N_KEYS, TOK_BLK), 0)
    for h in range(HEADS):
        q1 = qT[h * K_DIM:h * K_DIM + HALF, :]
        q2 = qT[h * K_DIM + HALF:(h + 1) * K_DIM, :]
        s1 = jnp.dot(keys_ref[2 * h], q1, preferred_element_type=jnp.float32)
        s2 = jnp.dot(keys_ref[2 * h + 1], q2,
                     preferred_element_type=jnp.float32)
        p1 = _topk16v(_pack(s1, iota))
        p2 = _topk16v(_pack(s2, iota))
        i1 = p1 & 255
        i2 = p2 & 255
        v1 = _unpack(p1)
        v2 = _unpack(p2)
        comb = (v1[:, None, :] + v2[None, :, :]).reshape(KNN * KNN, TOK_BLK)
        pc = _topk16v(_pack(comb, iota))
        c = pc & 255
        v = _unpack(pc)
        m = jnp.max(v, axis=0, keepdims=True)
        e = jnp.exp(v - m)
        w = e / jnp.sum(e, axis=0, keepdims=True)
        fin = _rowpick(c >> 4, i1) * N_KEYS + _rowpick(c & 15, i2)
        idxT_ref[:, h * KNN:(h + 1) * KNN] = fin.T
        wT_ref[:, h * KNN:(h + 1) * KNN] = w.T


def _out_body(x_ref, bag_ref, wswT_ref, bsw_ref, wvpT_ref, bvp_ref, o_ref):
    x = x_ref[...]
    g = jnp.dot(x, wswT_ref[...], preferred_element_type=jnp.float32)
    g = g + bsw_ref[...]
    g = g / (1.0 + jnp.exp(-g)) * bag_ref[...]
    o_ref[...] = jnp.dot(g, wvpT_ref[...],
                         preferred_element_type=jnp.float32) + bvp_ref[...]


JU = 4


def _accum_chunk(q, buf, w_v, out_v):
    tok = q >> 1
    row = tok & (OUTB - 1)
    r = q & 1

    def cg_body(cg, carry):
        accs0 = tuple(out_v[row, pl.ds(cg * 256 + ci * 16, 16)]
                      for ci in range(16))

        def j_body(jj, accs):
            accs = list(accs)
            for u in range(JU):
                j = jj * JU + u
                wchunk = w_v[tok, pl.ds(r * QROWS + (j >> 4) * 16, 16)]
                sp = _splat(wchunk, j & 15)
                for ci in range(16):
                    accs[ci] = accs[ci] + sp * buf[
                        j, pl.ds(cg * 256 + ci * 16, 16)]
            return tuple(accs)

        accs = lax.fori_loop(0, QROWS // JU, j_body, accs0)
        for ci in range(16):
            out_v[row, pl.ds(cg * 256 + ci * 16, 16)] = accs[ci]
        return carry

    lax.fori_loop(0, V_DIM // 256, cg_body, 0)


def _zero_out(out_v):
    def zero_body(t, carry):
        for ci in range(V_DIM // 16):
            out_v[t, pl.ds(ci * 16, 16)] = jnp.zeros((16,), jnp.float32)
        return carry

    lax.fori_loop(0, OUTB, zero_body, 0)


def _make_bag_body(tpw, nq):
    def _bag_body(values_hbm, idx_hbm, w_hbm, out_hbm, idx_v, w_v, bufa,
                  bufb, out_v, sema, semb):
        cid = lax.axis_index("c")
        sid = lax.axis_index("s")
        wid = sid * 2 + cid
        base = wid * tpw
        pltpu.sync_copy(idx_hbm.at[pl.ds(base * QPT, nq)], idx_v)
        pltpu.sync_copy(w_hbm.at[pl.ds(base, tpw)], w_v)
        _zero_out(out_v)

        pltpu.make_async_copy(values_hbm.at[idx_v.at[0]], bufa, sema).start()
        pltpu.make_async_copy(values_hbm.at[idx_v.at[1]], bufb, semb).start()

        def pair_body(qq, carry):
            q0 = qq * 2
            q1 = q0 + 1
            n0 = lax.rem(q0 + 2, nq)
            n1 = lax.rem(q1 + 2, nq)
            pltpu.make_async_copy(values_hbm.at[idx_v.at[q0]], bufa,
                                  sema).wait()
            _accum_chunk(q0, bufa, w_v, out_v)
            pltpu.make_async_copy(values_hbm.at[idx_v.at[n0]], bufa,
                                  sema).start()
            pltpu.make_async_copy(values_hbm.at[idx_v.at[q1]], bufb,
                                  semb).wait()
            _accum_chunk(q1, bufb, w_v, out_v)
            pltpu.make_async_copy(values_hbm.at[idx_v.at[n1]], bufb,
                                  semb).start()

            @pl.when(lax.rem(qq, OUTB) == OUTB - 1)
            def _flush():
                off = pl.multiple_of(base + qq - (OUTB - 1), OUTB)
                pltpu.sync_copy(out_v, out_hbm.at[pl.ds(off, OUTB)])
                _zero_out(out_v)

            return carry

        lax.fori_loop(0, nq // 2, pair_body, 0)
        pltpu.make_async_copy(values_hbm.at[idx_v.at[0]], bufa, sema).wait()
        pltpu.make_async_copy(values_hbm.at[idx_v.at[1]], bufb, semb).wait()

    return _bag_body


def _bag_call(values, idxq, w, ntok):
    tpw = ntok // NWORK
    nq = tpw * QPT
    mesh = plsc.VectorSubcoreMesh(core_axis_name="c", subcore_axis_name="s")
    return pl.kernel(
        _make_bag_body(tpw, nq),
        mesh=mesh,
        out_type=jax.ShapeDtypeStruct((ntok, V_DIM), jnp.float32),
        scratch_types=[
            pltpu.VMEM((nq, QROWS), jnp.int32),
            pltpu.VMEM((tpw, NSEL), jnp.float32),
            pltpu.VMEM((QROWS, V_DIM), jnp.float32),
            pltpu.VMEM((QROWS, V_DIM), jnp.float32),
            pltpu.VMEM((OUTB, V_DIM), jnp.float32),
            pltpu.SemaphoreType.DMA,
            pltpu.SemaphoreType.DMA,
        ],
    )(values, idxq, w)


def _select_call(x, Wq, bq2, keysN, ntok):
    grid = ntok // TOK_BLK
    return pl.pallas_call(
        _select_body,
        grid=(grid,),
        in_specs=[
            pl.BlockSpec((TOK_BLK, INPUT_DIM), lambda i: (i, 0)),
            pl.BlockSpec((HEADS * K_DIM, INPUT_DIM), lambda i: (0, 0)),
            pl.BlockSpec((HEADS * K_DIM, 1), lambda i: (0, 0)),
            pl.BlockSpec((2 * HEADS, N_KEYS, HALF), lambda i: (0, 0, 0)),
        ],
        out_specs=[
            pl.BlockSpec((TOK_BLK, NSEL), lambda i: (i, 0)),
            pl.BlockSpec((TOK_BLK, NSEL), lambda i: (i, 0)),
        ],
        out_shape=[
            jax.ShapeDtypeStruct((ntok, NSEL), jnp.int32),
            jax.ShapeDtypeStruct((ntok, NSEL), jnp.float32),
        ],
    )(x, Wq, bq2, keysN)


def _out_call(x, bag, WswT, bsw2, WvpT, bvp2, ntok):
    grid = ntok // TOK_BLK
    return pl.pallas_call(
        _out_body,
        grid=(grid,),
        in_specs=[
            pl.BlockSpec((TOK_BLK, INPUT_DIM), lambda i: (i, 0)),
            pl.BlockSpec((TOK_BLK, V_DIM), lambda i: (i, 0)),
            pl.BlockSpec((INPUT_DIM, V_DIM), lambda i: (0, 0)),
            pl.BlockSpec((1, V_DIM), lambda i: (0, 0)),
            pl.BlockSpec((V_DIM, OUTPUT_DIM), lambda i: (0, 0)),
            pl.BlockSpec((1, OUTPUT_DIM), lambda i: (0, 0)),
        ],
        out_specs=pl.BlockSpec((TOK_BLK, OUTPUT_DIM), lambda i: (i, 0)),
        out_shape=jax.ShapeDtypeStruct((ntok, OUTPUT_DIM), jnp.float32),
    )(x, bag, WswT, bsw2, WvpT, bvp2)


@jax.jit
def kernel(x, keys, values, Wq, bq, Wvp, bvp, Wsw, bsw):
    keysN = keys.reshape(2 * HEADS, N_KEYS, HALF)
    bq2 = bq.reshape(HEADS * K_DIM, 1)
    WswT = Wsw.T
    WvpT = Wvp.T
    bsw2 = bsw.reshape(1, V_DIM)
    bvp2 = bvp.reshape(1, OUTPUT_DIM)
    htok = TOKENS // 2

    outs = []
    bags = [None, None]
    idxqs = [None, None]
    ws = [None, None]
    for hf in range(2):
        idx, wsel = _select_call(
            x[hf * htok:(hf + 1) * htok], Wq, bq2, keysN, htok)
        idxqs[hf] = idx.reshape(htok * NSEL // QROWS, QROWS)
        ws[hf] = wsel
    for hf in range(2):
        bags[hf] = _bag_call(values, idxqs[hf], ws[hf], htok)
    for hf in range(2):
        outs.append(_out_call(
            x[hf * htok:(hf + 1) * htok], bags[hf], WswT, bsw2, WvpT, bvp2,
            htok))
    return jnp.concatenate(outs, axis=0)
